# baseline (speedup 1.0000x reference)
import jax
import jax.numpy as jnp
from jax.experimental import pallas as pl
from jax.experimental.pallas import tpu as pltpu
from jax.experimental.pallas import tpu_sc as plsc

_T, _D, _E, _K, _I = 2048, 1024, 8, 2, 512
_SI = 512
_BLK = 128
_NPAD = _T * _K + _E * _BLK
_NBLK = _NPAD // _BLK
_TB = 256
_W = 128
_F = 2


def _routing_kernel(x_ref, gw_ref, i0_ref, i1_ref, w0_ref, w1_ref, xbf_ref):
    xb = x_ref[...]
    l = jnp.dot(xb, gw_ref[...], preferred_element_type=jnp.float32)
    lane = jax.lax.broadcasted_iota(jnp.int32, l.shape, 1)
    neg = jnp.float32(-1e30)
    l = jnp.where(lane < _E, l, neg)
    m0 = jnp.max(l, axis=1, keepdims=True)
    i0 = jnp.min(jnp.where(l == m0, lane, _E), axis=1, keepdims=True)
    l1 = jnp.where(lane == i0, neg, l)
    m1 = jnp.max(l1, axis=1, keepdims=True)
    i1 = jnp.min(jnp.where(l1 == m1, lane, _E), axis=1, keepdims=True)
    e1 = jnp.exp(m1 - m0)
    s = 1.0 + e1
    i0_ref[...] = jnp.broadcast_to(i0, i0_ref.shape)
    i1_ref[...] = jnp.broadcast_to(i1, i1_ref.shape)
    w0_ref[...] = jnp.broadcast_to(1.0 / s, w0_ref.shape)
    w1_ref[...] = jnp.broadcast_to(e1 / s, w1_ref.shape)
    xbf_ref[...] = xb.astype(jnp.bfloat16)


def _wcast_kernel(wgu_ref, wd_ref, bgu_ref, bd_ref):
    bgu_ref[...] = wgu_ref[...].astype(jnp.bfloat16)
    bd_ref[...] = wd_ref[...].astype(jnp.bfloat16)


def _moe_mlp_kernel(bexp_ref, xs_ref, wgu_ref, wd_ref, y_ref):
    del bexp_ref
    gu = jnp.dot(xs_ref[...], wgu_ref[0], preferred_element_type=jnp.float32)
    g = gu[:, :_I]
    u = gu[:, _I:]
    a = (g * jax.nn.sigmoid(g) * u).astype(jnp.bfloat16)
    y_ref[...] = jnp.dot(a, wd_ref[0], preferred_element_type=jnp.float32
                         ).astype(jnp.bfloat16)


def _shared_mlp_kernel(x_ref, wgu_ref, wd_ref, o_ref):
    xb = x_ref[...].astype(jnp.bfloat16)
    gu = jnp.dot(xb, wgu_ref[...], preferred_element_type=jnp.float32)
    g = gu[:, :_SI]
    u = gu[:, _SI:]
    a = (g * jax.nn.sigmoid(g) * u).astype(jnp.bfloat16)
    o_ref[...] = jnp.dot(a, wd_ref[...], preferred_element_type=jnp.float32)


def _combine_kernel(g_ref, sh_ref, w0_ref, w1_ref, o_ref):
    o_ref[...] = (w0_ref[:, 0:1] * g_ref[0].astype(jnp.float32)
                  + w1_ref[:, 0:1] * g_ref[1].astype(jnp.float32)
                  + sh_ref[...])


def _sc_gather(data, idx, n, d):
    if data.dtype == jnp.bfloat16:
        d32 = d // 2
        data = jax.lax.bitcast_convert_type(
            data.reshape(-1, d32, 2), jnp.int32)
        unpack = True
    else:
        d32 = d
        unpack = False
    d2 = d32 // _F
    n2 = n * _F
    idx2 = (idx[:, None] * _F
            + jnp.arange(_F, dtype=idx.dtype)[None, :]).reshape(1, n2)
    data2 = data.reshape(-1, d2)
    mesh = plsc.VectorSubcoreMesh(core_axis_name="core",
                                  subcore_axis_name="subcore")

    @pl.kernel(out_type=jax.ShapeDtypeStruct((n2, d2), data2.dtype), mesh=mesh)
    def k(x_hbm, i_hbm, o_hbm):
        def body(i_vmem, o_vmem):
            pltpu.sync_copy(x_hbm.at[i_vmem.at[0]], o_vmem)

        pltpu.emit_pipeline(
            body,
            grid=(n2 // _W,),
            in_specs=[pl.BlockSpec((1, _W), lambda i: (0, i))],
            out_specs=[pl.BlockSpec((_W, d2), lambda i: (i, 0))],
            core_axis_name=("core", "subcore"),
            dimension_semantics=(pltpu.PARALLEL,),
        )(i_hbm, o_hbm)

    out = k(data2, idx2)
    if unpack:
        out = jax.lax.bitcast_convert_type(out.reshape(n, d32), jnp.bfloat16)
    return out.reshape(n, d)


def kernel(hidden_states, gate_w, w_gate_up, w_down, sh_gate_up, sh_down):
    x = hidden_states.reshape(_T, _D)
    gwp = jnp.pad(gate_w, ((0, 0), (0, 128 - _E)))

    i0b, i1b, w0b, w1b, xbf = pl.pallas_call(
        _routing_kernel,
        grid=(_T // _TB,),
        in_specs=[pl.BlockSpec((_TB, _D), lambda i: (i, 0)),
                  pl.BlockSpec((_D, 128), lambda i: (0, 0))],
        out_specs=[pl.BlockSpec((_TB, 128), lambda i: (i, 0))] * 4
        + [pl.BlockSpec((_TB, _D), lambda i: (i, 0))],
        out_shape=[jax.ShapeDtypeStruct((_T, 128), jnp.int32),
                   jax.ShapeDtypeStruct((_T, 128), jnp.int32),
                   jax.ShapeDtypeStruct((_T, 128), jnp.float32),
                   jax.ShapeDtypeStruct((_T, 128), jnp.float32),
                   jax.ShapeDtypeStruct((_T, _D), jnp.bfloat16)],
    )(x, gwp)

    bgu, bd = pl.pallas_call(
        _wcast_kernel,
        grid=(_E,),
        in_specs=[pl.BlockSpec((1, _D, 2 * _I), lambda i: (i, 0, 0)),
                  pl.BlockSpec((1, _I, _D), lambda i: (i, 0, 0))],
        out_specs=[pl.BlockSpec((1, _D, 2 * _I), lambda i: (i, 0, 0)),
                   pl.BlockSpec((1, _I, _D), lambda i: (i, 0, 0))],
        out_shape=[jax.ShapeDtypeStruct((_E, _D, 2 * _I), jnp.bfloat16),
                   jax.ShapeDtypeStruct((_E, _I, _D), jnp.bfloat16)],
    )(w_gate_up, w_down)
    sbgu, sbd = pl.pallas_call(
        _wcast_kernel,
        grid=(1,),
        in_specs=[pl.BlockSpec((_D, 2 * _SI), lambda i: (0, 0)),
                  pl.BlockSpec((_SI, _D), lambda i: (0, 0))],
        out_specs=[pl.BlockSpec((_D, 2 * _SI), lambda i: (0, 0)),
                   pl.BlockSpec((_SI, _D), lambda i: (0, 0))],
        out_shape=[jax.ShapeDtypeStruct((_D, 2 * _SI), jnp.bfloat16),
                   jax.ShapeDtypeStruct((_SI, _D), jnp.bfloat16)],
    )(sh_gate_up, sh_down)

    i0 = i0b[:, 0]
    i1 = i1b[:, 0]
    e_flat = jnp.concatenate([i0, i1])
    toks = jnp.concatenate([jnp.arange(_T, dtype=jnp.int32)] * 2)
    oh = (e_flat[:, None] == jnp.arange(_E, dtype=jnp.int32)[None, :])
    csum = jnp.cumsum(oh.astype(jnp.int32), axis=0)
    counts = csum[-1]
    rank = jnp.take_along_axis(csum, e_flat[:, None], axis=1)[:, 0] - 1
    padded = ((counts + _BLK - 1) // _BLK) * _BLK
    ends = jnp.cumsum(padded)
    offs = ends - padded
    r = offs[e_flat] + rank
    src = jnp.zeros((_NPAD,), jnp.int32).at[r].set(toks, unique_indices=True)
    bstart = jnp.arange(_NBLK, dtype=jnp.int32) * _BLK
    bexp = jnp.minimum(jnp.searchsorted(ends, bstart, side="right"),
                       _E - 1).astype(jnp.int32)

    xs = _sc_gather(xbf, src, _NPAD, _D)

    y = pl.pallas_call(
        _moe_mlp_kernel,
        grid_spec=pltpu.PrefetchScalarGridSpec(
            num_scalar_prefetch=1,
            grid=(_NBLK,),
            in_specs=[pl.BlockSpec((_BLK, _D), lambda i, b: (i, 0)),
                      pl.BlockSpec((1, _D, 2 * _I), lambda i, b: (b[i], 0, 0)),
                      pl.BlockSpec((1, _I, _D), lambda i, b: (b[i], 0, 0))],
            out_specs=pl.BlockSpec((_BLK, _D), lambda i, b: (i, 0)),
        ),
        out_shape=jax.ShapeDtypeStruct((_NPAD, _D), jnp.bfloat16),
        compiler_params=pltpu.CompilerParams(
            dimension_semantics=("arbitrary",)),
    )(bexp, xs, bgu, bd)

    sh = pl.pallas_call(
        _shared_mlp_kernel,
        grid=(_T // _TB,),
        in_specs=[pl.BlockSpec((_TB, _D), lambda i: (i, 0)),
                  pl.BlockSpec((_D, 2 * _SI), lambda i: (0, 0)),
                  pl.BlockSpec((_SI, _D), lambda i: (0, 0))],
        out_specs=pl.BlockSpec((_TB, _D), lambda i: (i, 0)),
        out_shape=jax.ShapeDtypeStruct((_T, _D), jnp.float32),
    )(x, sbgu, sbd)

    g2 = _sc_gather(y, r, _K * _T, _D).reshape(_K, _T, _D)

    final = pl.pallas_call(
        _combine_kernel,
        grid=(_T // _TB,),
        in_specs=[pl.BlockSpec((_K, _TB, _D), lambda i: (0, i, 0)),
                  pl.BlockSpec((_TB, _D), lambda i: (i, 0)),
                  pl.BlockSpec((_TB, 128), lambda i: (i, 0)),
                  pl.BlockSpec((_TB, 128), lambda i: (i, 0))],
        out_specs=pl.BlockSpec((_TB, _D), lambda i: (i, 0)),
        out_shape=jax.ShapeDtypeStruct((_T, _D), jnp.float32),
    )(g2, sh, w0b, w1b)
    return final.reshape(hidden_states.shape)


# speedup vs baseline: 1.8876x; 1.8876x over previous
import jax
import jax.numpy as jnp
from jax.experimental import pallas as pl
from jax.experimental.pallas import tpu as pltpu
from jax.experimental.pallas import tpu_sc as plsc

_T, _D, _E, _K, _I = 2048, 1024, 8, 2, 512
_SI = 512
_BLK = 128
_NPAD = _T * _K + _E * _BLK
_NBLK = _NPAD // _BLK
_TB = 256
_W = 128
_F = 4


def _routing_kernel(x_ref, gw_ref, i0_ref, i1_ref, w0_ref, w1_ref):
    xb = x_ref[...]
    l = jnp.dot(xb, gw_ref[...], preferred_element_type=jnp.float32)
    lane = jax.lax.broadcasted_iota(jnp.int32, l.shape, 1)
    neg = jnp.float32(-1e30)
    l = jnp.where(lane < _E, l, neg)
    m0 = jnp.max(l, axis=1, keepdims=True)
    i0 = jnp.min(jnp.where(l == m0, lane, _E), axis=1, keepdims=True)
    l1 = jnp.where(lane == i0, neg, l)
    m1 = jnp.max(l1, axis=1, keepdims=True)
    i1 = jnp.min(jnp.where(l1 == m1, lane, _E), axis=1, keepdims=True)
    e1 = jnp.exp(m1 - m0)
    s = 1.0 + e1
    i0_ref[...] = jnp.broadcast_to(i0, i0_ref.shape)
    i1_ref[...] = jnp.broadcast_to(i1, i1_ref.shape)
    w0_ref[...] = jnp.broadcast_to(1.0 / s, w0_ref.shape)
    w1_ref[...] = jnp.broadcast_to(e1 / s, w1_ref.shape)


def _wcast_kernel(wgu_ref, wd_ref, bgu_ref, bd_ref):
    bgu_ref[...] = wgu_ref[...].astype(jnp.bfloat16)
    bd_ref[...] = wd_ref[...].astype(jnp.bfloat16)


def _moe_mlp_kernel(bexp_ref, xs_ref, wgu_ref, wd_ref, y_ref):
    del bexp_ref
    xb = xs_ref[...].astype(jnp.bfloat16)
    gu = jnp.dot(xb, wgu_ref[0], preferred_element_type=jnp.float32)
    g = gu[:, :_I]
    u = gu[:, _I:]
    a = (g * jax.nn.sigmoid(g) * u).astype(jnp.bfloat16)
    y_ref[...] = jnp.dot(a, wd_ref[0], preferred_element_type=jnp.float32)


def _shared_mlp_kernel(x_ref, wgu_ref, wd_ref, o_ref):
    xb = x_ref[...].astype(jnp.bfloat16)
    gu = jnp.dot(xb, wgu_ref[...], preferred_element_type=jnp.float32)
    g = gu[:, :_SI]
    u = gu[:, _SI:]
    a = (g * jax.nn.sigmoid(g) * u).astype(jnp.bfloat16)
    o_ref[...] = jnp.dot(a, wd_ref[...], preferred_element_type=jnp.float32)


def _combine_kernel(g_ref, sh_ref, w0_ref, w1_ref, o_ref):
    o_ref[...] = (w0_ref[:, 0:1] * g_ref[0]
                  + w1_ref[:, 0:1] * g_ref[1]
                  + sh_ref[...])


def _sc_gather(data, idx, n, d):
    d2 = d // _F
    n2 = n * _F
    idx2 = (idx[:, None] * _F
            + jnp.arange(_F, dtype=idx.dtype)[None, :]).reshape(1, n2)
    data2 = data.reshape(-1, d2)
    mesh = plsc.VectorSubcoreMesh(core_axis_name="core",
                                  subcore_axis_name="subcore")

    @pl.kernel(out_type=jax.ShapeDtypeStruct((n2, d2), data2.dtype), mesh=mesh)
    def k(x_hbm, i_hbm, o_hbm):
        def body(i_vmem, o_vmem):
            pltpu.sync_copy(x_hbm.at[i_vmem.at[0]], o_vmem)

        pltpu.emit_pipeline(
            body,
            grid=(n2 // _W,),
            in_specs=[pl.BlockSpec((1, _W), lambda i: (0, i))],
            out_specs=[pl.BlockSpec((_W, d2), lambda i: (i, 0))],
            core_axis_name=("core", "subcore"),
            dimension_semantics=(pltpu.PARALLEL,),
        )(i_hbm, o_hbm)

    return k(data2, idx2).reshape(n, d)


def kernel(hidden_states, gate_w, w_gate_up, w_down, sh_gate_up, sh_down):
    x = hidden_states.reshape(_T, _D)
    gwp = jnp.pad(gate_w, ((0, 0), (0, 128 - _E)))

    i0b, i1b, w0b, w1b = pl.pallas_call(
        _routing_kernel,
        grid=(_T // _TB,),
        in_specs=[pl.BlockSpec((_TB, _D), lambda i: (i, 0)),
                  pl.BlockSpec((_D, 128), lambda i: (0, 0))],
        out_specs=[pl.BlockSpec((_TB, 128), lambda i: (i, 0))] * 4,
        out_shape=[jax.ShapeDtypeStruct((_T, 128), jnp.int32),
                   jax.ShapeDtypeStruct((_T, 128), jnp.int32),
                   jax.ShapeDtypeStruct((_T, 128), jnp.float32),
                   jax.ShapeDtypeStruct((_T, 128), jnp.float32)],
    )(x, gwp)

    bgu, bd = pl.pallas_call(
        _wcast_kernel,
        grid=(_E,),
        in_specs=[pl.BlockSpec((1, _D, 2 * _I), lambda i: (i, 0, 0)),
                  pl.BlockSpec((1, _I, _D), lambda i: (i, 0, 0))],
        out_specs=[pl.BlockSpec((1, _D, 2 * _I), lambda i: (i, 0, 0)),
                   pl.BlockSpec((1, _I, _D), lambda i: (i, 0, 0))],
        out_shape=[jax.ShapeDtypeStruct((_E, _D, 2 * _I), jnp.bfloat16),
                   jax.ShapeDtypeStruct((_E, _I, _D), jnp.bfloat16)],
    )(w_gate_up, w_down)
    sbgu, sbd = pl.pallas_call(
        _wcast_kernel,
        grid=(1,),
        in_specs=[pl.BlockSpec((_D, 2 * _SI), lambda i: (0, 0)),
                  pl.BlockSpec((_SI, _D), lambda i: (0, 0))],
        out_specs=[pl.BlockSpec((_D, 2 * _SI), lambda i: (0, 0)),
                   pl.BlockSpec((_SI, _D), lambda i: (0, 0))],
        out_shape=[jax.ShapeDtypeStruct((_D, 2 * _SI), jnp.bfloat16),
                   jax.ShapeDtypeStruct((_SI, _D), jnp.bfloat16)],
    )(sh_gate_up, sh_down)

    i0 = i0b[:, 0]
    i1 = i1b[:, 0]
    e_flat = jnp.concatenate([i0, i1])
    toks = jnp.concatenate([jnp.arange(_T, dtype=jnp.int32)] * 2)
    oh = (e_flat[:, None] == jnp.arange(_E, dtype=jnp.int32)[None, :])
    csum = jnp.cumsum(oh.astype(jnp.int32), axis=0)
    counts = csum[-1]
    rank = jnp.take_along_axis(csum, e_flat[:, None], axis=1)[:, 0] - 1
    padded = ((counts + _BLK - 1) // _BLK) * _BLK
    ends = jnp.cumsum(padded)
    offs = ends - padded
    r = offs[e_flat] + rank
    src = jnp.zeros((_NPAD,), jnp.int32).at[r].set(toks, unique_indices=True)
    bstart = jnp.arange(_NBLK, dtype=jnp.int32) * _BLK
    bexp = jnp.minimum(jnp.searchsorted(ends, bstart, side="right"),
                       _E - 1).astype(jnp.int32)

    xs = _sc_gather(x, src, _NPAD, _D)

    y = pl.pallas_call(
        _moe_mlp_kernel,
        grid_spec=pltpu.PrefetchScalarGridSpec(
            num_scalar_prefetch=1,
            grid=(_NBLK,),
            in_specs=[pl.BlockSpec((_BLK, _D), lambda i, b: (i, 0)),
                      pl.BlockSpec((1, _D, 2 * _I), lambda i, b: (b[i], 0, 0)),
                      pl.BlockSpec((1, _I, _D), lambda i, b: (b[i], 0, 0))],
            out_specs=pl.BlockSpec((_BLK, _D), lambda i, b: (i, 0)),
        ),
        out_shape=jax.ShapeDtypeStruct((_NPAD, _D), jnp.float32),
        compiler_params=pltpu.CompilerParams(
            dimension_semantics=("arbitrary",)),
    )(bexp, xs, bgu, bd)

    sh = pl.pallas_call(
        _shared_mlp_kernel,
        grid=(_T // _TB,),
        in_specs=[pl.BlockSpec((_TB, _D), lambda i: (i, 0)),
                  pl.BlockSpec((_D, 2 * _SI), lambda i: (0, 0)),
                  pl.BlockSpec((_SI, _D), lambda i: (0, 0))],
        out_specs=pl.BlockSpec((_TB, _D), lambda i: (i, 0)),
        out_shape=jax.ShapeDtypeStruct((_T, _D), jnp.float32),
    )(x, sbgu, sbd)

    g2 = _sc_gather(y, r, _K * _T, _D).reshape(_K, _T, _D)

    final = pl.pallas_call(
        _combine_kernel,
        grid=(_T // _TB,),
        in_specs=[pl.BlockSpec((_K, _TB, _D), lambda i: (0, i, 0)),
                  pl.BlockSpec((_TB, _D), lambda i: (i, 0)),
                  pl.BlockSpec((_TB, 128), lambda i: (i, 0)),
                  pl.BlockSpec((_TB, 128), lambda i: (i, 0))],
        out_specs=pl.BlockSpec((_TB, _D), lambda i: (i, 0)),
        out_shape=jax.ShapeDtypeStruct((_T, _D), jnp.float32),
    )(g2, sh, w0b, w1b)
    return final.reshape(hidden_states.shape)


# speedup vs baseline: 2.4615x; 1.3040x over previous
import functools

import jax
import jax.numpy as jnp
from jax import lax
from jax.experimental import pallas as pl
from jax.experimental.pallas import tpu as pltpu
from jax.experimental.pallas import tpu_sc as plsc

_T, _D, _E, _K, _I = 2048, 1024, 8, 2, 512
_SI = 512
_BLK = 128
_NPAD = _T * _K + _E * _BLK
_NBLK = _NPAD // _BLK
_TB = 256
_W = 128
_F = 4


def _routing_kernel(x_ref, gw_ref, i0_ref, i1_ref, w0_ref, w1_ref):
    xb = x_ref[...]
    l = jnp.dot(xb, gw_ref[...], preferred_element_type=jnp.float32)
    lane = jax.lax.broadcasted_iota(jnp.int32, l.shape, 1)
    neg = jnp.float32(-1e30)
    l = jnp.where(lane < _E, l, neg)
    m0 = jnp.max(l, axis=1, keepdims=True)
    i0 = jnp.min(jnp.where(l == m0, lane, _E), axis=1, keepdims=True)
    l1 = jnp.where(lane == i0, neg, l)
    m1 = jnp.max(l1, axis=1, keepdims=True)
    i1 = jnp.min(jnp.where(l1 == m1, lane, _E), axis=1, keepdims=True)
    e1 = jnp.exp(m1 - m0)
    s = 1.0 + e1
    i0_ref[...] = jnp.broadcast_to(i0, i0_ref.shape)
    i1_ref[...] = jnp.broadcast_to(i1, i1_ref.shape)
    w0_ref[...] = jnp.broadcast_to(1.0 / s, w0_ref.shape)
    w1_ref[...] = jnp.broadcast_to(e1 / s, w1_ref.shape)


def _wcast_kernel(wgu_ref, wd_ref, bgu_ref, bd_ref):
    bgu_ref[...] = wgu_ref[...].astype(jnp.bfloat16)
    bd_ref[...] = wd_ref[...].astype(jnp.bfloat16)


def _moe_mlp_kernel(bexp_ref, xs_ref, wgu_ref, wd_ref, y_ref):
    del bexp_ref
    xb = xs_ref[...].astype(jnp.bfloat16)
    gu = jnp.dot(xb, wgu_ref[0], preferred_element_type=jnp.float32)
    g = gu[:, :_I]
    u = gu[:, _I:]
    a = (g * jax.nn.sigmoid(g) * u).astype(jnp.bfloat16)
    y_ref[...] = jnp.dot(a, wd_ref[0], preferred_element_type=jnp.float32)


def _shared_mlp_kernel(x_ref, wgu_ref, wd_ref, o_ref):
    xb = x_ref[...].astype(jnp.bfloat16)
    gu = jnp.dot(xb, wgu_ref[...], preferred_element_type=jnp.float32)
    g = gu[:, :_SI]
    u = gu[:, _SI:]
    a = (g * jax.nn.sigmoid(g) * u).astype(jnp.bfloat16)
    o_ref[...] = jnp.dot(a, wd_ref[...], preferred_element_type=jnp.float32)


def _combine_kernel(g_ref, sh_ref, w0_ref, w1_ref, o_ref):
    o_ref[...] = (w0_ref[:, 0:1] * g_ref[0]
                  + w1_ref[:, 0:1] * g_ref[1]
                  + sh_ref[...])


_GC = 32
_NW = 32


def _sc_gather(data, idx, n, d):
    per = n // _NW
    nch = per // _GC
    idx3 = idx.reshape(_NW, nch, _GC)
    mesh = plsc.VectorSubcoreMesh(core_axis_name="c", subcore_axis_name="s")

    @functools.partial(
        pl.kernel, mesh=mesh,
        out_type=jax.ShapeDtypeStruct((n, d), data.dtype),
        scratch_types=[pltpu.VMEM((nch, _GC), jnp.int32),
                       pltpu.VMEM((_GC, d), data.dtype),
                       pltpu.VMEM((_GC, d), data.dtype),
                       pltpu.SemaphoreType.DMA,
                       pltpu.SemaphoreType.DMA,
                       pltpu.SemaphoreType.DMA,
                       pltpu.SemaphoreType.DMA])
    def k(x_hbm, i_hbm, o_hbm, idx_v, b0, b1, g0, g1, o0, o1):
        bufs = (b0, b1)
        gsem = (g0, g1)
        osem = (o0, o1)
        wid = lax.axis_index("s") * 2 + lax.axis_index("c")
        pltpu.sync_copy(i_hbm.at[wid], idx_v)
        base = wid * (nch * _GC)
        g = [None] * nch
        o = [None] * nch
        g[0] = pltpu.async_copy(x_hbm.at[idx_v.at[0]], bufs[0], gsem[0])
        for j in range(nch):
            nx = j + 1
            if nx < nch:
                if nx >= 2:
                    o[nx - 2].wait()
                g[nx] = pltpu.async_copy(
                    x_hbm.at[idx_v.at[nx]], bufs[nx % 2], gsem[nx % 2])
            g[j].wait()
            o[j] = pltpu.async_copy(
                bufs[j % 2], o_hbm.at[pl.ds(base + j * _GC, _GC)],
                osem[j % 2])
        if nch >= 2:
            o[nch - 2].wait()
        o[nch - 1].wait()

    return k(data, idx3)


def kernel(hidden_states, gate_w, w_gate_up, w_down, sh_gate_up, sh_down):
    x = hidden_states.reshape(_T, _D)
    gwp = jnp.pad(gate_w, ((0, 0), (0, 128 - _E)))

    i0b, i1b, w0b, w1b = pl.pallas_call(
        _routing_kernel,
        grid=(_T // _TB,),
        in_specs=[pl.BlockSpec((_TB, _D), lambda i: (i, 0)),
                  pl.BlockSpec((_D, 128), lambda i: (0, 0))],
        out_specs=[pl.BlockSpec((_TB, 128), lambda i: (i, 0))] * 4,
        out_shape=[jax.ShapeDtypeStruct((_T, 128), jnp.int32),
                   jax.ShapeDtypeStruct((_T, 128), jnp.int32),
                   jax.ShapeDtypeStruct((_T, 128), jnp.float32),
                   jax.ShapeDtypeStruct((_T, 128), jnp.float32)],
    )(x, gwp)

    bgu, bd = pl.pallas_call(
        _wcast_kernel,
        grid=(_E,),
        in_specs=[pl.BlockSpec((1, _D, 2 * _I), lambda i: (i, 0, 0)),
                  pl.BlockSpec((1, _I, _D), lambda i: (i, 0, 0))],
        out_specs=[pl.BlockSpec((1, _D, 2 * _I), lambda i: (i, 0, 0)),
                   pl.BlockSpec((1, _I, _D), lambda i: (i, 0, 0))],
        out_shape=[jax.ShapeDtypeStruct((_E, _D, 2 * _I), jnp.bfloat16),
                   jax.ShapeDtypeStruct((_E, _I, _D), jnp.bfloat16)],
    )(w_gate_up, w_down)
    sbgu, sbd = pl.pallas_call(
        _wcast_kernel,
        grid=(1,),
        in_specs=[pl.BlockSpec((_D, 2 * _SI), lambda i: (0, 0)),
                  pl.BlockSpec((_SI, _D), lambda i: (0, 0))],
        out_specs=[pl.BlockSpec((_D, 2 * _SI), lambda i: (0, 0)),
                   pl.BlockSpec((_SI, _D), lambda i: (0, 0))],
        out_shape=[jax.ShapeDtypeStruct((_D, 2 * _SI), jnp.bfloat16),
                   jax.ShapeDtypeStruct((_SI, _D), jnp.bfloat16)],
    )(sh_gate_up, sh_down)

    i0 = i0b[:, 0]
    i1 = i1b[:, 0]
    e_flat = jnp.concatenate([i0, i1])
    toks = jnp.concatenate([jnp.arange(_T, dtype=jnp.int32)] * 2)
    oh = (e_flat[:, None] == jnp.arange(_E, dtype=jnp.int32)[None, :])
    csum = jnp.cumsum(oh.astype(jnp.int32), axis=0)
    counts = csum[-1]
    rank = jnp.take_along_axis(csum, e_flat[:, None], axis=1)[:, 0] - 1
    padded = ((counts + _BLK - 1) // _BLK) * _BLK
    ends = jnp.cumsum(padded)
    offs = ends - padded
    r = offs[e_flat] + rank
    src = jnp.zeros((_NPAD,), jnp.int32).at[r].set(toks, unique_indices=True)
    bstart = jnp.arange(_NBLK, dtype=jnp.int32) * _BLK
    bexp = jnp.minimum(jnp.searchsorted(ends, bstart, side="right"),
                       _E - 1).astype(jnp.int32)

    xs = _sc_gather(x, src, _NPAD, _D)

    y = pl.pallas_call(
        _moe_mlp_kernel,
        grid_spec=pltpu.PrefetchScalarGridSpec(
            num_scalar_prefetch=1,
            grid=(_NBLK,),
            in_specs=[pl.BlockSpec((_BLK, _D), lambda i, b: (i, 0)),
                      pl.BlockSpec((1, _D, 2 * _I), lambda i, b: (b[i], 0, 0)),
                      pl.BlockSpec((1, _I, _D), lambda i, b: (b[i], 0, 0))],
            out_specs=pl.BlockSpec((_BLK, _D), lambda i, b: (i, 0)),
        ),
        out_shape=jax.ShapeDtypeStruct((_NPAD, _D), jnp.float32),
        compiler_params=pltpu.CompilerParams(
            dimension_semantics=("arbitrary",)),
    )(bexp, xs, bgu, bd)

    sh = pl.pallas_call(
        _shared_mlp_kernel,
        grid=(_T // _TB,),
        in_specs=[pl.BlockSpec((_TB, _D), lambda i: (i, 0)),
                  pl.BlockSpec((_D, 2 * _SI), lambda i: (0, 0)),
                  pl.BlockSpec((_SI, _D), lambda i: (0, 0))],
        out_specs=pl.BlockSpec((_TB, _D), lambda i: (i, 0)),
        out_shape=jax.ShapeDtypeStruct((_T, _D), jnp.float32),
    )(x, sbgu, sbd)

    g2 = _sc_gather(y, r, _K * _T, _D).reshape(_K, _T, _D)

    final = pl.pallas_call(
        _combine_kernel,
        grid=(_T // _TB,),
        in_specs=[pl.BlockSpec((_K, _TB, _D), lambda i: (0, i, 0)),
                  pl.BlockSpec((_TB, _D), lambda i: (i, 0)),
                  pl.BlockSpec((_TB, 128), lambda i: (i, 0)),
                  pl.BlockSpec((_TB, 128), lambda i: (i, 0))],
        out_specs=pl.BlockSpec((_TB, _D), lambda i: (i, 0)),
        out_shape=jax.ShapeDtypeStruct((_T, _D), jnp.float32),
    )(g2, sh, w0b, w1b)
    return final.reshape(hidden_states.shape)


# speedup vs baseline: 3.9322x; 1.5975x over previous
import functools

import jax
import jax.numpy as jnp
from jax import lax
from jax.experimental import pallas as pl
from jax.experimental.pallas import tpu as pltpu
from jax.experimental.pallas import tpu_sc as plsc

_T, _D, _E, _K, _I = 2048, 1024, 8, 2, 512
_SI = 512
_BLK = 128
_NPAD = _T * _K + _E * _BLK
_NBLK = _NPAD // _BLK
_TB = 256
_GC = 32
_NW = 32


def _routing_kernel(x_ref, gw_ref, i0_ref, i1_ref, w0_ref, w1_ref,
                    r0_ref, r1_ref, cnt_ref, cacc_ref):
    @pl.when(pl.program_id(0) == 0)
    def _():
        cacc_ref[...] = jnp.zeros_like(cacc_ref)

    l = jnp.dot(x_ref[...], gw_ref[...], preferred_element_type=jnp.float32)
    lane = jax.lax.broadcasted_iota(jnp.int32, l.shape, 1)
    neg = jnp.float32(-1e30)
    l = jnp.where(lane < _E, l, neg)
    m0 = jnp.max(l, axis=1, keepdims=True)
    i0 = jnp.min(jnp.where(l == m0, lane, _E), axis=1, keepdims=True)
    l1 = jnp.where(lane == i0, neg, l)
    m1 = jnp.max(l1, axis=1, keepdims=True)
    i1 = jnp.min(jnp.where(l1 == m1, lane, _E), axis=1, keepdims=True)
    e1 = jnp.exp(m1 - m0)
    s = 1.0 + e1

    oh0 = (lane == i0).astype(jnp.int32)
    oh1 = (lane == i1).astype(jnp.int32)
    row = jax.lax.broadcasted_iota(jnp.int32, (_TB, _TB), 0)
    col = jax.lax.broadcasted_iota(jnp.int32, (_TB, _TB), 1)
    tri = (row >= col).astype(jnp.float32)
    c0 = jnp.dot(tri, oh0.astype(jnp.float32),
                 preferred_element_type=jnp.float32).astype(jnp.int32)
    c1 = jnp.dot(tri, oh1.astype(jnp.float32),
                 preferred_element_type=jnp.float32).astype(jnp.int32)
    tot0 = c0[-1:, :]
    tot1 = c1[-1:, :]
    cnt = cacc_ref[...]
    rank0 = jnp.sum(jnp.where(oh0 > 0, cnt + c0 - 1, 0), axis=1,
                    keepdims=True)
    rank1 = jnp.sum(jnp.where(oh1 > 0, cnt + tot0 + c1 - 1, 0), axis=1,
                    keepdims=True)
    newcnt = cnt + tot0 + tot1
    cacc_ref[...] = newcnt
    cnt_ref[...] = newcnt

    i0_ref[...] = jnp.broadcast_to(i0, i0_ref.shape)
    i1_ref[...] = jnp.broadcast_to(i1, i1_ref.shape)
    w0_ref[...] = jnp.broadcast_to(1.0 / s, w0_ref.shape)
    w1_ref[...] = jnp.broadcast_to(e1 / s, w1_ref.shape)
    r0_ref[...] = jnp.broadcast_to(rank0, r0_ref.shape)
    r1_ref[...] = jnp.broadcast_to(rank1, r1_ref.shape)


def _wcast_kernel(wgu_ref, wd_ref, bgu_ref, bd_ref):
    bgu_ref[...] = wgu_ref[...].astype(jnp.bfloat16)
    bd_ref[...] = wd_ref[...].astype(jnp.bfloat16)


def _moe_mlp_kernel(bexp_ref, xs_ref, wgu_ref, wd_ref, y_ref):
    del bexp_ref
    xb = xs_ref[...].astype(jnp.bfloat16)
    gu = jnp.dot(xb, wgu_ref[0], preferred_element_type=jnp.float32)
    g = gu[:, :_I]
    u = gu[:, _I:]
    a = (g * jax.nn.sigmoid(g) * u).astype(jnp.bfloat16)
    y_ref[...] = jnp.dot(a, wd_ref[0], preferred_element_type=jnp.float32)


def _shared_mlp_kernel(x_ref, wgu_ref, wd_ref, o_ref):
    xb = x_ref[...].astype(jnp.bfloat16)
    gu = jnp.dot(xb, wgu_ref[...], preferred_element_type=jnp.float32)
    g = gu[:, :_SI]
    u = gu[:, _SI:]
    a = (g * jax.nn.sigmoid(g) * u).astype(jnp.bfloat16)
    o_ref[...] = jnp.dot(a, wd_ref[...], preferred_element_type=jnp.float32)


def _combine_kernel(g_ref, sh_ref, w0_ref, w1_ref, o_ref):
    o_ref[...] = (w0_ref[:, 0:1] * g_ref[0]
                  + w1_ref[:, 0:1] * g_ref[1]
                  + sh_ref[...])


def _sc_dispatch(x, r0, r1):
    per = _T // _NW
    i0_3 = r0.reshape(_NW, 1, per)
    i1_3 = r1.reshape(_NW, 1, per)
    mesh = plsc.VectorSubcoreMesh(core_axis_name="c", subcore_axis_name="s")

    @functools.partial(
        pl.kernel, mesh=mesh,
        out_type=jax.ShapeDtypeStruct((_NPAD, _D), x.dtype),
        scratch_types=[pltpu.VMEM((1, per), jnp.int32),
                       pltpu.VMEM((1, per), jnp.int32),
                       pltpu.VMEM((per, _D), x.dtype),
                       pltpu.SemaphoreType.DMA,
                       pltpu.SemaphoreType.DMA])
    def k(x_hbm, i0_hbm, i1_hbm, o_hbm, idx0_v, idx1_v, buf, s0, s1):
        wid = lax.axis_index("s") * 2 + lax.axis_index("c")
        pltpu.sync_copy(i0_hbm.at[wid], idx0_v)
        pltpu.sync_copy(i1_hbm.at[wid], idx1_v)
        pltpu.sync_copy(x_hbm.at[pl.ds(wid * per, per)], buf)
        c0 = pltpu.async_copy(buf, o_hbm.at[idx0_v.at[0]], s0)
        c1 = pltpu.async_copy(buf, o_hbm.at[idx1_v.at[0]], s1)
        c0.wait()
        c1.wait()

    return k(x, i0_3, i1_3)


def _sc_gather(data, idx, n, d):
    per = n // _NW
    nch = per // _GC
    idx3 = idx.reshape(_NW, nch, _GC)
    mesh = plsc.VectorSubcoreMesh(core_axis_name="c", subcore_axis_name="s")

    @functools.partial(
        pl.kernel, mesh=mesh,
        out_type=jax.ShapeDtypeStruct((n, d), data.dtype),
        scratch_types=[pltpu.VMEM((nch, _GC), jnp.int32),
                       pltpu.VMEM((_GC, d), data.dtype),
                       pltpu.VMEM((_GC, d), data.dtype),
                       pltpu.SemaphoreType.DMA,
                       pltpu.SemaphoreType.DMA,
                       pltpu.SemaphoreType.DMA,
                       pltpu.SemaphoreType.DMA])
    def k(x_hbm, i_hbm, o_hbm, idx_v, b0, b1, g0, g1, o0, o1):
        bufs = (b0, b1)
        gsem = (g0, g1)
        osem = (o0, o1)
        wid = lax.axis_index("s") * 2 + lax.axis_index("c")
        pltpu.sync_copy(i_hbm.at[wid], idx_v)
        base = wid * (nch * _GC)
        g = [None] * nch
        o = [None] * nch
        g[0] = pltpu.async_copy(x_hbm.at[idx_v.at[0]], bufs[0], gsem[0])
        for j in range(nch):
            nx = j + 1
            if nx < nch:
                if nx >= 2:
                    o[nx - 2].wait()
                g[nx] = pltpu.async_copy(
                    x_hbm.at[idx_v.at[nx]], bufs[nx % 2], gsem[nx % 2])
            g[j].wait()
            o[j] = pltpu.async_copy(
                bufs[j % 2], o_hbm.at[pl.ds(base + j * _GC, _GC)],
                osem[j % 2])
        if nch >= 2:
            o[nch - 2].wait()
        o[nch - 1].wait()

    return k(data, idx3)


def kernel(hidden_states, gate_w, w_gate_up, w_down, sh_gate_up, sh_down):
    x = hidden_states.reshape(_T, _D)
    gwp = jnp.pad(gate_w, ((0, 0), (0, 128 - _E)))

    i0b, i1b, w0b, w1b, r0b, r1b, cntb = pl.pallas_call(
        _routing_kernel,
        grid=(_T // _TB,),
        in_specs=[pl.BlockSpec((_TB, _D), lambda i: (i, 0)),
                  pl.BlockSpec((_D, 128), lambda i: (0, 0))],
        out_specs=[pl.BlockSpec((_TB, 128), lambda i: (i, 0))] * 6
        + [pl.BlockSpec((1, 128), lambda i: (0, 0))],
        out_shape=[jax.ShapeDtypeStruct((_T, 128), jnp.int32),
                   jax.ShapeDtypeStruct((_T, 128), jnp.int32),
                   jax.ShapeDtypeStruct((_T, 128), jnp.float32),
                   jax.ShapeDtypeStruct((_T, 128), jnp.float32),
                   jax.ShapeDtypeStruct((_T, 128), jnp.int32),
                   jax.ShapeDtypeStruct((_T, 128), jnp.int32),
                   jax.ShapeDtypeStruct((1, 128), jnp.int32)],
        scratch_shapes=[pltpu.VMEM((1, 128), jnp.int32)],
    )(x, gwp)

    bgu, bd = pl.pallas_call(
        _wcast_kernel,
        grid=(_E,),
        in_specs=[pl.BlockSpec((1, _D, 2 * _I), lambda i: (i, 0, 0)),
                  pl.BlockSpec((1, _I, _D), lambda i: (i, 0, 0))],
        out_specs=[pl.BlockSpec((1, _D, 2 * _I), lambda i: (i, 0, 0)),
                   pl.BlockSpec((1, _I, _D), lambda i: (i, 0, 0))],
        out_shape=[jax.ShapeDtypeStruct((_E, _D, 2 * _I), jnp.bfloat16),
                   jax.ShapeDtypeStruct((_E, _I, _D), jnp.bfloat16)],
    )(w_gate_up, w_down)
    sbgu, sbd = pl.pallas_call(
        _wcast_kernel,
        grid=(1,),
        in_specs=[pl.BlockSpec((_D, 2 * _SI), lambda i: (0, 0)),
                  pl.BlockSpec((_SI, _D), lambda i: (0, 0))],
        out_specs=[pl.BlockSpec((_D, 2 * _SI), lambda i: (0, 0)),
                   pl.BlockSpec((_SI, _D), lambda i: (0, 0))],
        out_shape=[jax.ShapeDtypeStruct((_D, 2 * _SI), jnp.bfloat16),
                   jax.ShapeDtypeStruct((_SI, _D), jnp.bfloat16)],
    )(sh_gate_up, sh_down)

    counts = cntb[0, :_E]
    padded = ((counts + _BLK - 1) // _BLK) * _BLK
    ends = jnp.cumsum(padded)
    offs = ends - padded
    eidx = jnp.arange(_E, dtype=jnp.int32)
    i0 = i0b[:, 0]
    i1 = i1b[:, 0]
    off0 = jnp.sum(jnp.where(i0[:, None] == eidx[None, :],
                             offs[None, :], 0), axis=1)
    off1 = jnp.sum(jnp.where(i1[:, None] == eidx[None, :],
                             offs[None, :], 0), axis=1)
    r0 = r0b[:, 0] + off0.astype(jnp.int32)
    r1 = r1b[:, 0] + off1.astype(jnp.int32)
    bstart = jnp.arange(_NBLK, dtype=jnp.int32)[:, None] * _BLK
    bexp = jnp.minimum(
        jnp.sum((ends[None, :] <= bstart).astype(jnp.int32), axis=1),
        _E - 1).astype(jnp.int32)

    xs = _sc_dispatch(x, r0, r1)

    y = pl.pallas_call(
        _moe_mlp_kernel,
        grid_spec=pltpu.PrefetchScalarGridSpec(
            num_scalar_prefetch=1,
            grid=(_NBLK,),
            in_specs=[pl.BlockSpec((_BLK, _D), lambda i, b: (i, 0)),
                      pl.BlockSpec((1, _D, 2 * _I), lambda i, b: (b[i], 0, 0)),
                      pl.BlockSpec((1, _I, _D), lambda i, b: (b[i], 0, 0))],
            out_specs=pl.BlockSpec((_BLK, _D), lambda i, b: (i, 0)),
        ),
        out_shape=jax.ShapeDtypeStruct((_NPAD, _D), jnp.float32),
        compiler_params=pltpu.CompilerParams(
            dimension_semantics=("arbitrary",)),
    )(bexp, xs, bgu, bd)

    sh = pl.pallas_call(
        _shared_mlp_kernel,
        grid=(_T // _TB,),
        in_specs=[pl.BlockSpec((_TB, _D), lambda i: (i, 0)),
                  pl.BlockSpec((_D, 2 * _SI), lambda i: (0, 0)),
                  pl.BlockSpec((_SI, _D), lambda i: (0, 0))],
        out_specs=pl.BlockSpec((_TB, _D), lambda i: (i, 0)),
        out_shape=jax.ShapeDtypeStruct((_T, _D), jnp.float32),
    )(x, sbgu, sbd)

    ridx = jnp.concatenate([r0, r1])
    g2 = _sc_gather(y, ridx, _K * _T, _D).reshape(_K, _T, _D)

    final = pl.pallas_call(
        _combine_kernel,
        grid=(_T // _TB,),
        in_specs=[pl.BlockSpec((_K, _TB, _D), lambda i: (0, i, 0)),
                  pl.BlockSpec((_TB, _D), lambda i: (i, 0)),
                  pl.BlockSpec((_TB, 128), lambda i: (i, 0)),
                  pl.BlockSpec((_TB, 128), lambda i: (i, 0))],
        out_specs=pl.BlockSpec((_TB, _D), lambda i: (i, 0)),
        out_shape=jax.ShapeDtypeStruct((_T, _D), jnp.float32),
    )(g2, sh, w0b, w1b)
    return final.reshape(hidden_states.shape)


# speedup vs baseline: 4.4111x; 1.1218x over previous
import functools

import jax
import jax.numpy as jnp
from jax import lax
from jax.experimental import pallas as pl
from jax.experimental.pallas import tpu as pltpu
from jax.experimental.pallas import tpu_sc as plsc

_T, _D, _E, _K, _I = 2048, 1024, 8, 2, 512
_SI = 512
_BLK = 128
_NPAD = _T * _K + _E * _BLK
_NBLK = _NPAD // _BLK
_TB = 256
_GC = 32
_NW = 32


def _routing_kernel(x_ref, gw_ref, i0_ref, i1_ref, w0_ref, w1_ref,
                    r0_ref, r1_ref, cnt_ref, cacc_ref):
    @pl.when(pl.program_id(0) == 0)
    def _():
        cacc_ref[...] = jnp.zeros_like(cacc_ref)

    l = jnp.dot(x_ref[...], gw_ref[...], preferred_element_type=jnp.float32)
    lane = jax.lax.broadcasted_iota(jnp.int32, l.shape, 1)
    neg = jnp.float32(-1e30)
    l = jnp.where(lane < _E, l, neg)
    m0 = jnp.max(l, axis=1, keepdims=True)
    i0 = jnp.min(jnp.where(l == m0, lane, _E), axis=1, keepdims=True)
    l1 = jnp.where(lane == i0, neg, l)
    m1 = jnp.max(l1, axis=1, keepdims=True)
    i1 = jnp.min(jnp.where(l1 == m1, lane, _E), axis=1, keepdims=True)
    e1 = jnp.exp(m1 - m0)
    s = 1.0 + e1

    oh0 = (lane == i0).astype(jnp.int32)
    oh1 = (lane == i1).astype(jnp.int32)
    row = jax.lax.broadcasted_iota(jnp.int32, (_TB, _TB), 0)
    col = jax.lax.broadcasted_iota(jnp.int32, (_TB, _TB), 1)
    tri = (row >= col).astype(jnp.float32)
    c0 = jnp.dot(tri, oh0.astype(jnp.float32),
                 preferred_element_type=jnp.float32).astype(jnp.int32)
    c1 = jnp.dot(tri, oh1.astype(jnp.float32),
                 preferred_element_type=jnp.float32).astype(jnp.int32)
    tot0 = c0[-1:, :]
    tot1 = c1[-1:, :]
    cnt = cacc_ref[...]
    rank0 = jnp.sum(jnp.where(oh0 > 0, cnt + c0 - 1, 0), axis=1,
                    keepdims=True)
    rank1 = jnp.sum(jnp.where(oh1 > 0, cnt + tot0 + c1 - 1, 0), axis=1,
                    keepdims=True)
    newcnt = cnt + tot0 + tot1
    cacc_ref[...] = newcnt
    cnt_ref[...] = newcnt

    i0_ref[...] = jnp.broadcast_to(i0, i0_ref.shape)
    i1_ref[...] = jnp.broadcast_to(i1, i1_ref.shape)
    w0_ref[...] = jnp.broadcast_to(1.0 / s, w0_ref.shape)
    w1_ref[...] = jnp.broadcast_to(e1 / s, w1_ref.shape)
    r0_ref[...] = jnp.broadcast_to(rank0, r0_ref.shape)
    r1_ref[...] = jnp.broadcast_to(rank1, r1_ref.shape)


def _moe_mlp_kernel(bexp_ref, xs_ref, wgu_ref, wd_ref, y_ref,
                    bgu_ref, bd_ref):
    i = pl.program_id(0)
    changed = jnp.logical_or(
        i == 0, bexp_ref[i] != bexp_ref[jnp.maximum(i - 1, 0)])

    @pl.when(changed)
    def _():
        bgu_ref[...] = wgu_ref[0].astype(jnp.bfloat16)
        bd_ref[...] = wd_ref[0].astype(jnp.bfloat16)

    xb = xs_ref[...].astype(jnp.bfloat16)
    gu = jnp.dot(xb, bgu_ref[...], preferred_element_type=jnp.float32)
    g = gu[:, :_I]
    u = gu[:, _I:]
    a = (g * jax.nn.sigmoid(g) * u).astype(jnp.bfloat16)
    y_ref[...] = jnp.dot(a, bd_ref[...], preferred_element_type=jnp.float32)


def _shared_mlp_kernel(x_ref, wgu_ref, wd_ref, o_ref, bgu_ref, bd_ref):
    @pl.when(pl.program_id(0) == 0)
    def _():
        bgu_ref[...] = wgu_ref[...].astype(jnp.bfloat16)
        bd_ref[...] = wd_ref[...].astype(jnp.bfloat16)

    xb = x_ref[...].astype(jnp.bfloat16)
    gu = jnp.dot(xb, bgu_ref[...], preferred_element_type=jnp.float32)
    g = gu[:, :_SI]
    u = gu[:, _SI:]
    a = (g * jax.nn.sigmoid(g) * u).astype(jnp.bfloat16)
    o_ref[...] = jnp.dot(a, bd_ref[...], preferred_element_type=jnp.float32)


def _combine_kernel(g_ref, sh_ref, w0_ref, w1_ref, o_ref):
    o_ref[...] = (w0_ref[:, 0:1] * g_ref[0]
                  + w1_ref[:, 0:1] * g_ref[1]
                  + sh_ref[...])


def _sc_dispatch(x, r0, r1):
    per = _T // _NW
    i0_3 = r0.reshape(_NW, 1, per)
    i1_3 = r1.reshape(_NW, 1, per)
    mesh = plsc.VectorSubcoreMesh(core_axis_name="c", subcore_axis_name="s")

    @functools.partial(
        pl.kernel, mesh=mesh,
        out_type=jax.ShapeDtypeStruct((_NPAD, _D), x.dtype),
        scratch_types=[pltpu.VMEM((1, per), jnp.int32),
                       pltpu.VMEM((1, per), jnp.int32),
                       pltpu.VMEM((per, _D), x.dtype),
                       pltpu.SemaphoreType.DMA,
                       pltpu.SemaphoreType.DMA])
    def k(x_hbm, i0_hbm, i1_hbm, o_hbm, idx0_v, idx1_v, buf, s0, s1):
        wid = lax.axis_index("s") * 2 + lax.axis_index("c")
        pltpu.sync_copy(i0_hbm.at[wid], idx0_v)
        pltpu.sync_copy(i1_hbm.at[wid], idx1_v)
        pltpu.sync_copy(x_hbm.at[pl.ds(wid * per, per)], buf)
        c0 = pltpu.async_copy(buf, o_hbm.at[idx0_v.at[0]], s0)
        c1 = pltpu.async_copy(buf, o_hbm.at[idx1_v.at[0]], s1)
        c0.wait()
        c1.wait()

    return k(x, i0_3, i1_3)


def _sc_gather(data, idx, n, d):
    per = n // _NW
    nch = per // _GC
    idx3 = idx.reshape(_NW, nch, _GC)
    mesh = plsc.VectorSubcoreMesh(core_axis_name="c", subcore_axis_name="s")

    @functools.partial(
        pl.kernel, mesh=mesh,
        out_type=jax.ShapeDtypeStruct((n, d), data.dtype),
        scratch_types=[pltpu.VMEM((nch, _GC), jnp.int32),
                       pltpu.VMEM((_GC, d), data.dtype),
                       pltpu.VMEM((_GC, d), data.dtype),
                       pltpu.SemaphoreType.DMA,
                       pltpu.SemaphoreType.DMA,
                       pltpu.SemaphoreType.DMA,
                       pltpu.SemaphoreType.DMA])
    def k(x_hbm, i_hbm, o_hbm, idx_v, b0, b1, g0, g1, o0, o1):
        bufs = (b0, b1)
        gsem = (g0, g1)
        osem = (o0, o1)
        wid = lax.axis_index("s") * 2 + lax.axis_index("c")
        pltpu.sync_copy(i_hbm.at[wid], idx_v)
        base = wid * (nch * _GC)
        g = [None] * nch
        o = [None] * nch
        g[0] = pltpu.async_copy(x_hbm.at[idx_v.at[0]], bufs[0], gsem[0])
        for j in range(nch):
            nx = j + 1
            if nx < nch:
                if nx >= 2:
                    o[nx - 2].wait()
                g[nx] = pltpu.async_copy(
                    x_hbm.at[idx_v.at[nx]], bufs[nx % 2], gsem[nx % 2])
            g[j].wait()
            o[j] = pltpu.async_copy(
                bufs[j % 2], o_hbm.at[pl.ds(base + j * _GC, _GC)],
                osem[j % 2])
        if nch >= 2:
            o[nch - 2].wait()
        o[nch - 1].wait()

    return k(data, idx3)


def kernel(hidden_states, gate_w, w_gate_up, w_down, sh_gate_up, sh_down):
    x = hidden_states.reshape(_T, _D)
    gwp = jnp.pad(gate_w, ((0, 0), (0, 128 - _E)))

    i0b, i1b, w0b, w1b, r0b, r1b, cntb = pl.pallas_call(
        _routing_kernel,
        grid=(_T // _TB,),
        in_specs=[pl.BlockSpec((_TB, _D), lambda i: (i, 0)),
                  pl.BlockSpec((_D, 128), lambda i: (0, 0))],
        out_specs=[pl.BlockSpec((_TB, 128), lambda i: (i, 0))] * 6
        + [pl.BlockSpec((1, 128), lambda i: (0, 0))],
        out_shape=[jax.ShapeDtypeStruct((_T, 128), jnp.int32),
                   jax.ShapeDtypeStruct((_T, 128), jnp.int32),
                   jax.ShapeDtypeStruct((_T, 128), jnp.float32),
                   jax.ShapeDtypeStruct((_T, 128), jnp.float32),
                   jax.ShapeDtypeStruct((_T, 128), jnp.int32),
                   jax.ShapeDtypeStruct((_T, 128), jnp.int32),
                   jax.ShapeDtypeStruct((1, 128), jnp.int32)],
        scratch_shapes=[pltpu.VMEM((1, 128), jnp.int32)],
    )(x, gwp)

    counts = cntb[0, :_E]
    padded = ((counts + _BLK - 1) // _BLK) * _BLK
    ends = jnp.cumsum(padded)
    offs = ends - padded
    eidx = jnp.arange(_E, dtype=jnp.int32)
    i0 = i0b[:, 0]
    i1 = i1b[:, 0]
    off0 = jnp.sum(jnp.where(i0[:, None] == eidx[None, :],
                             offs[None, :], 0), axis=1)
    off1 = jnp.sum(jnp.where(i1[:, None] == eidx[None, :],
                             offs[None, :], 0), axis=1)
    r0 = r0b[:, 0] + off0.astype(jnp.int32)
    r1 = r1b[:, 0] + off1.astype(jnp.int32)
    bstart = jnp.arange(_NBLK, dtype=jnp.int32)[:, None] * _BLK
    bexp = jnp.minimum(
        jnp.sum((ends[None, :] <= bstart).astype(jnp.int32), axis=1),
        _E - 1).astype(jnp.int32)

    xs = _sc_dispatch(x, r0, r1)

    y = pl.pallas_call(
        _moe_mlp_kernel,
        grid_spec=pltpu.PrefetchScalarGridSpec(
            num_scalar_prefetch=1,
            grid=(_NBLK,),
            in_specs=[pl.BlockSpec((_BLK, _D), lambda i, b: (i, 0)),
                      pl.BlockSpec((1, _D, 2 * _I), lambda i, b: (b[i], 0, 0)),
                      pl.BlockSpec((1, _I, _D), lambda i, b: (b[i], 0, 0))],
            out_specs=pl.BlockSpec((_BLK, _D), lambda i, b: (i, 0)),
            scratch_shapes=[pltpu.VMEM((_D, 2 * _I), jnp.bfloat16),
                            pltpu.VMEM((_I, _D), jnp.bfloat16)],
        ),
        out_shape=jax.ShapeDtypeStruct((_NPAD, _D), jnp.float32),
        compiler_params=pltpu.CompilerParams(
            dimension_semantics=("arbitrary",)),
    )(bexp, xs, w_gate_up, w_down)

    sh = pl.pallas_call(
        _shared_mlp_kernel,
        grid=(_T // _TB,),
        in_specs=[pl.BlockSpec((_TB, _D), lambda i: (i, 0)),
                  pl.BlockSpec((_D, 2 * _SI), lambda i: (0, 0)),
                  pl.BlockSpec((_SI, _D), lambda i: (0, 0))],
        out_specs=pl.BlockSpec((_TB, _D), lambda i: (i, 0)),
        out_shape=jax.ShapeDtypeStruct((_T, _D), jnp.float32),
        scratch_shapes=[pltpu.VMEM((_D, 2 * _SI), jnp.bfloat16),
                        pltpu.VMEM((_SI, _D), jnp.bfloat16)],
    )(x, sh_gate_up, sh_down)

    ridx = jnp.concatenate([r0, r1])
    g2 = _sc_gather(y, ridx, _K * _T, _D).reshape(_K, _T, _D)

    final = pl.pallas_call(
        _combine_kernel,
        grid=(_T // _TB,),
        in_specs=[pl.BlockSpec((_K, _TB, _D), lambda i: (0, i, 0)),
                  pl.BlockSpec((_TB, _D), lambda i: (i, 0)),
                  pl.BlockSpec((_TB, 128), lambda i: (i, 0)),
                  pl.BlockSpec((_TB, 128), lambda i: (i, 0))],
        out_specs=pl.BlockSpec((_TB, _D), lambda i: (i, 0)),
        out_shape=jax.ShapeDtypeStruct((_T, _D), jnp.float32),
    )(g2, sh, w0b, w1b)
    return final.reshape(hidden_states.shape)


# speedup vs baseline: 4.7573x; 1.0785x over previous
import functools

import jax
import jax.numpy as jnp
from jax import lax
from jax.experimental import pallas as pl
from jax.experimental.pallas import tpu as pltpu
from jax.experimental.pallas import tpu_sc as plsc

_T, _D, _E, _K, _I = 2048, 1024, 8, 2, 512
_SI = 512
_BLK = 128
_NPAD = _T * _K + _E * _BLK
_NBLK = _NPAD // _BLK
_TB = 256
_GC = 32
_NW = 32


def _pack_bf16(v):
    m = v.shape[1] // 2
    r = v.astype(jnp.bfloat16).astype(jnp.float32)
    bits = jax.lax.bitcast_convert_type(r, jnp.int32)
    lo = jax.lax.shift_right_logical(bits[:, :m], 16)
    hi = jnp.bitwise_and(bits[:, m:], jnp.int32(-65536))
    return jnp.bitwise_or(hi, lo)


def _unpack_bf16(v):
    lo = jax.lax.bitcast_convert_type(
        jax.lax.shift_left(v, 16), jnp.float32)
    hi = jax.lax.bitcast_convert_type(
        jnp.bitwise_and(v, jnp.int32(-65536)), jnp.float32)
    return jnp.concatenate([lo, hi], axis=1)


def _routing_kernel(x_ref, gw_ref, i0_ref, i1_ref, w0_ref, w1_ref,
                    r0_ref, r1_ref, cnt_ref, xp_ref, cacc_ref):
    @pl.when(pl.program_id(0) == 0)
    def _():
        cacc_ref[...] = jnp.zeros_like(cacc_ref)

    xp_ref[...] = _pack_bf16(x_ref[...])
    l = jnp.dot(x_ref[...], gw_ref[...], preferred_element_type=jnp.float32)
    lane = jax.lax.broadcasted_iota(jnp.int32, l.shape, 1)
    neg = jnp.float32(-1e30)
    l = jnp.where(lane < _E, l, neg)
    m0 = jnp.max(l, axis=1, keepdims=True)
    i0 = jnp.min(jnp.where(l == m0, lane, _E), axis=1, keepdims=True)
    l1 = jnp.where(lane == i0, neg, l)
    m1 = jnp.max(l1, axis=1, keepdims=True)
    i1 = jnp.min(jnp.where(l1 == m1, lane, _E), axis=1, keepdims=True)
    e1 = jnp.exp(m1 - m0)
    s = 1.0 + e1

    oh0 = (lane == i0).astype(jnp.int32)
    oh1 = (lane == i1).astype(jnp.int32)
    row = jax.lax.broadcasted_iota(jnp.int32, (_TB, _TB), 0)
    col = jax.lax.broadcasted_iota(jnp.int32, (_TB, _TB), 1)
    tri = (row >= col).astype(jnp.float32)
    c0 = jnp.dot(tri, oh0.astype(jnp.float32),
                 preferred_element_type=jnp.float32).astype(jnp.int32)
    c1 = jnp.dot(tri, oh1.astype(jnp.float32),
                 preferred_element_type=jnp.float32).astype(jnp.int32)
    tot0 = c0[-1:, :]
    tot1 = c1[-1:, :]
    cnt = cacc_ref[...]
    rank0 = jnp.sum(jnp.where(oh0 > 0, cnt + c0 - 1, 0), axis=1,
                    keepdims=True)
    rank1 = jnp.sum(jnp.where(oh1 > 0, cnt + tot0 + c1 - 1, 0), axis=1,
                    keepdims=True)
    newcnt = cnt + tot0 + tot1
    cacc_ref[...] = newcnt
    cnt_ref[...] = newcnt

    i0_ref[...] = jnp.broadcast_to(i0, i0_ref.shape)
    i1_ref[...] = jnp.broadcast_to(i1, i1_ref.shape)
    w0_ref[...] = jnp.broadcast_to(1.0 / s, w0_ref.shape)
    w1_ref[...] = jnp.broadcast_to(e1 / s, w1_ref.shape)
    r0_ref[...] = jnp.broadcast_to(rank0, r0_ref.shape)
    r1_ref[...] = jnp.broadcast_to(rank1, r1_ref.shape)


def _moe_mlp_kernel(bexp_ref, xs_ref, wgu_ref, wd_ref, y_ref,
                    bgu_ref, bd_ref):
    i = pl.program_id(0)
    changed = jnp.logical_or(
        i == 0, bexp_ref[i] != bexp_ref[jnp.maximum(i - 1, 0)])

    @pl.when(changed)
    def _():
        bgu_ref[...] = wgu_ref[0].astype(jnp.bfloat16)
        bd_ref[...] = wd_ref[0].astype(jnp.bfloat16)

    xb = _unpack_bf16(xs_ref[...]).astype(jnp.bfloat16)
    gu = jnp.dot(xb, bgu_ref[...], preferred_element_type=jnp.float32)
    g = gu[:, :_I]
    u = gu[:, _I:]
    a = (g * jax.nn.sigmoid(g) * u).astype(jnp.bfloat16)
    y_ref[...] = _pack_bf16(
        jnp.dot(a, bd_ref[...], preferred_element_type=jnp.float32))


def _shared_mlp_kernel(x_ref, wgu_ref, wd_ref, o_ref, bgu_ref, bd_ref):
    @pl.when(pl.program_id(0) == 0)
    def _():
        bgu_ref[...] = wgu_ref[...].astype(jnp.bfloat16)
        bd_ref[...] = wd_ref[...].astype(jnp.bfloat16)

    xb = x_ref[...].astype(jnp.bfloat16)
    gu = jnp.dot(xb, bgu_ref[...], preferred_element_type=jnp.float32)
    g = gu[:, :_SI]
    u = gu[:, _SI:]
    a = (g * jax.nn.sigmoid(g) * u).astype(jnp.bfloat16)
    o_ref[...] = jnp.dot(a, bd_ref[...], preferred_element_type=jnp.float32)


def _combine_kernel(g_ref, sh_ref, w0_ref, w1_ref, o_ref):
    o_ref[...] = (w0_ref[:, 0:1] * _unpack_bf16(g_ref[0])
                  + w1_ref[:, 0:1] * _unpack_bf16(g_ref[1])
                  + sh_ref[...])


def _sc_dispatch(x, r0, r1):
    per = _T // _NW
    d = x.shape[1]
    i0_3 = r0.reshape(_NW, 1, per)
    i1_3 = r1.reshape(_NW, 1, per)
    mesh = plsc.VectorSubcoreMesh(core_axis_name="c", subcore_axis_name="s")

    @functools.partial(
        pl.kernel, mesh=mesh,
        out_type=jax.ShapeDtypeStruct((_NPAD, d), x.dtype),
        scratch_types=[pltpu.VMEM((1, per), jnp.int32),
                       pltpu.VMEM((1, per), jnp.int32),
                       pltpu.VMEM((per, d), x.dtype),
                       pltpu.SemaphoreType.DMA,
                       pltpu.SemaphoreType.DMA])
    def k(x_hbm, i0_hbm, i1_hbm, o_hbm, idx0_v, idx1_v, buf, s0, s1):
        wid = lax.axis_index("s") * 2 + lax.axis_index("c")
        pltpu.sync_copy(i0_hbm.at[wid], idx0_v)
        pltpu.sync_copy(i1_hbm.at[wid], idx1_v)
        pltpu.sync_copy(x_hbm.at[pl.ds(wid * per, per)], buf)
        c0 = pltpu.async_copy(buf, o_hbm.at[idx0_v.at[0]], s0)
        c1 = pltpu.async_copy(buf, o_hbm.at[idx1_v.at[0]], s1)
        c0.wait()
        c1.wait()

    return k(x, i0_3, i1_3)


def _sc_gather(data, idx, n, d):
    per = n // _NW
    nch = per // _GC
    idx3 = idx.reshape(_NW, nch, _GC)
    mesh = plsc.VectorSubcoreMesh(core_axis_name="c", subcore_axis_name="s")

    @functools.partial(
        pl.kernel, mesh=mesh,
        out_type=jax.ShapeDtypeStruct((n, d), data.dtype),
        scratch_types=[pltpu.VMEM((nch, _GC), jnp.int32),
                       pltpu.VMEM((_GC, d), data.dtype),
                       pltpu.VMEM((_GC, d), data.dtype),
                       pltpu.SemaphoreType.DMA,
                       pltpu.SemaphoreType.DMA,
                       pltpu.SemaphoreType.DMA,
                       pltpu.SemaphoreType.DMA])
    def k(x_hbm, i_hbm, o_hbm, idx_v, b0, b1, g0, g1, o0, o1):
        bufs = (b0, b1)
        gsem = (g0, g1)
        osem = (o0, o1)
        wid = lax.axis_index("s") * 2 + lax.axis_index("c")
        pltpu.sync_copy(i_hbm.at[wid], idx_v)
        base = wid * (nch * _GC)
        g = [None] * nch
        o = [None] * nch
        g[0] = pltpu.async_copy(x_hbm.at[idx_v.at[0]], bufs[0], gsem[0])
        for j in range(nch):
            nx = j + 1
            if nx < nch:
                if nx >= 2:
                    o[nx - 2].wait()
                g[nx] = pltpu.async_copy(
                    x_hbm.at[idx_v.at[nx]], bufs[nx % 2], gsem[nx % 2])
            g[j].wait()
            o[j] = pltpu.async_copy(
                bufs[j % 2], o_hbm.at[pl.ds(base + j * _GC, _GC)],
                osem[j % 2])
        if nch >= 2:
            o[nch - 2].wait()
        o[nch - 1].wait()

    return k(data, idx3)


def kernel(hidden_states, gate_w, w_gate_up, w_down, sh_gate_up, sh_down):
    x = hidden_states.reshape(_T, _D)
    gwp = jnp.pad(gate_w, ((0, 0), (0, 128 - _E)))

    i0b, i1b, w0b, w1b, r0b, r1b, cntb, xp = pl.pallas_call(
        _routing_kernel,
        grid=(_T // _TB,),
        in_specs=[pl.BlockSpec((_TB, _D), lambda i: (i, 0)),
                  pl.BlockSpec((_D, 128), lambda i: (0, 0))],
        out_specs=[pl.BlockSpec((_TB, 128), lambda i: (i, 0))] * 6
        + [pl.BlockSpec((1, 128), lambda i: (0, 0)),
           pl.BlockSpec((_TB, _D // 2), lambda i: (i, 0))],
        out_shape=[jax.ShapeDtypeStruct((_T, 128), jnp.int32),
                   jax.ShapeDtypeStruct((_T, 128), jnp.int32),
                   jax.ShapeDtypeStruct((_T, 128), jnp.float32),
                   jax.ShapeDtypeStruct((_T, 128), jnp.float32),
                   jax.ShapeDtypeStruct((_T, 128), jnp.int32),
                   jax.ShapeDtypeStruct((_T, 128), jnp.int32),
                   jax.ShapeDtypeStruct((1, 128), jnp.int32),
                   jax.ShapeDtypeStruct((_T, _D // 2), jnp.int32)],
        scratch_shapes=[pltpu.VMEM((1, 128), jnp.int32)],
    )(x, gwp)

    counts = cntb[0, :_E]
    padded = ((counts + _BLK - 1) // _BLK) * _BLK
    ends = jnp.cumsum(padded)
    offs = ends - padded
    eidx = jnp.arange(_E, dtype=jnp.int32)
    i0 = i0b[:, 0]
    i1 = i1b[:, 0]
    off0 = jnp.sum(jnp.where(i0[:, None] == eidx[None, :],
                             offs[None, :], 0), axis=1)
    off1 = jnp.sum(jnp.where(i1[:, None] == eidx[None, :],
                             offs[None, :], 0), axis=1)
    r0 = r0b[:, 0] + off0.astype(jnp.int32)
    r1 = r1b[:, 0] + off1.astype(jnp.int32)
    bstart = jnp.arange(_NBLK, dtype=jnp.int32)[:, None] * _BLK
    bexp = jnp.minimum(
        jnp.sum((ends[None, :] <= bstart).astype(jnp.int32), axis=1),
        _E - 1).astype(jnp.int32)

    xs = _sc_dispatch(xp, r0, r1)

    y = pl.pallas_call(
        _moe_mlp_kernel,
        grid_spec=pltpu.PrefetchScalarGridSpec(
            num_scalar_prefetch=1,
            grid=(_NBLK,),
            in_specs=[pl.BlockSpec((_BLK, _D // 2), lambda i, b: (i, 0)),
                      pl.BlockSpec((1, _D, 2 * _I), lambda i, b: (b[i], 0, 0)),
                      pl.BlockSpec((1, _I, _D), lambda i, b: (b[i], 0, 0))],
            out_specs=pl.BlockSpec((_BLK, _D // 2), lambda i, b: (i, 0)),
            scratch_shapes=[pltpu.VMEM((_D, 2 * _I), jnp.bfloat16),
                            pltpu.VMEM((_I, _D), jnp.bfloat16)],
        ),
        out_shape=jax.ShapeDtypeStruct((_NPAD, _D // 2), jnp.int32),
        compiler_params=pltpu.CompilerParams(
            dimension_semantics=("arbitrary",)),
    )(bexp, xs, w_gate_up, w_down)

    sh = pl.pallas_call(
        _shared_mlp_kernel,
        grid=(_T // _TB,),
        in_specs=[pl.BlockSpec((_TB, _D), lambda i: (i, 0)),
                  pl.BlockSpec((_D, 2 * _SI), lambda i: (0, 0)),
                  pl.BlockSpec((_SI, _D), lambda i: (0, 0))],
        out_specs=pl.BlockSpec((_TB, _D), lambda i: (i, 0)),
        out_shape=jax.ShapeDtypeStruct((_T, _D), jnp.float32),
        scratch_shapes=[pltpu.VMEM((_D, 2 * _SI), jnp.bfloat16),
                        pltpu.VMEM((_SI, _D), jnp.bfloat16)],
    )(x, sh_gate_up, sh_down)

    ridx = jnp.concatenate([r0, r1])
    g2 = _sc_gather(y, ridx, _K * _T, _D // 2).reshape(_K, _T, _D // 2)

    final = pl.pallas_call(
        _combine_kernel,
        grid=(_T // _TB,),
        in_specs=[pl.BlockSpec((_K, _TB, _D // 2), lambda i: (0, i, 0)),
                  pl.BlockSpec((_TB, _D), lambda i: (i, 0)),
                  pl.BlockSpec((_TB, 128), lambda i: (i, 0)),
                  pl.BlockSpec((_TB, 128), lambda i: (i, 0))],
        out_specs=pl.BlockSpec((_TB, _D), lambda i: (i, 0)),
        out_shape=jax.ShapeDtypeStruct((_T, _D), jnp.float32),
    )(g2, sh, w0b, w1b)
    return final.reshape(hidden_states.shape)
